# baseline (speedup 1.0000x reference)
import functools

import jax
import jax.numpy as jnp
import numpy as np
from jax import lax
from jax.experimental import pallas as pl
from jax.experimental.pallas import tpu as pltpu
from jax.experimental.pallas import tpu_sc as plsc

_NNZ = 6400000
_SCALE = np.float32(1.0 / (1.0 - 0.2 + 1e-05))

_K1 = np.uint32(0)
_K2 = np.uint32(42)
_K3 = np.uint32(0x1BD11BDA) ^ _K1 ^ _K2

_THRESH = np.uint32(1677722 << 9)

_ROT0 = (13, 15, 26, 6)
_ROT1 = (17, 29, 16, 24)


def _rotl(x, d):
    return (x << np.uint32(d)) | (x >> np.uint32(32 - d))


def _threefry_bits(idx):
    x0 = jnp.zeros_like(idx) + _K1
    x1 = idx + _K2
    ks = (_K1, _K2, _K3)
    for r in range(5):
        rots = _ROT0 if r % 2 == 0 else _ROT1
        for d in rots:
            x0 = x0 + x1
            x1 = x0 ^ _rotl(x1, d)
        x0 = x0 + ks[(r + 1) % 3]
        x1 = x1 + ks[(r + 2) % 3] + np.uint32(r + 1)
    return x0 ^ x1


def _edge_drop_body(rows_per_blk, v_ref, o_ref):
    pid = pl.program_id(0)
    r = jax.lax.broadcasted_iota(jnp.uint32, (rows_per_blk, 128), 0)
    c = jax.lax.broadcasted_iota(jnp.uint32, (rows_per_blk, 128), 1)
    row0 = (pid * rows_per_blk).astype(jnp.uint32)
    idx = (row0 + r) * np.uint32(128) + c
    bits = _threefry_bits(idx)
    keep = bits >= _THRESH
    o_ref[...] = jnp.where(keep, v_ref[...] * _SCALE, jnp.float32(0.0))


_ICHUNK = 16000
_NCHUNK = _NNZ // _ICHUNK


def _sc_copy_body(i_hbm, o_hbm, buf):
    w = lax.axis_index("s") * 2 + lax.axis_index("c")
    start = w * 13 - jnp.maximum(w - 16, 0)
    n = 13 - (w >= 16).astype(jnp.int32)

    def body(i, carry):
        c0 = (start + i) * _ICHUNK
        pltpu.sync_copy(i_hbm.at[:, pl.ds(c0, _ICHUNK)], buf)
        pltpu.sync_copy(buf, o_hbm.at[:, pl.ds(c0, _ICHUNK)])
        return carry

    lax.fori_loop(0, n, body, 0)


def kernel(x_indices, x_values):
    rows = 50000
    rows_per_blk = 2000
    grid = rows // rows_per_blk
    v2d = x_values.reshape(rows, 128)

    sc_copy = pl.kernel(
        _sc_copy_body,
        out_type=jax.ShapeDtypeStruct((2, _NNZ), jnp.int32),
        mesh=plsc.VectorSubcoreMesh(core_axis_name="c", subcore_axis_name="s"),
        scratch_types=[pltpu.VMEM((2, _ICHUNK), jnp.int32)],
    )
    out_idx = sc_copy(x_indices)

    out = pl.pallas_call(
        functools.partial(_edge_drop_body, rows_per_blk),
        grid=(grid,),
        in_specs=[pl.BlockSpec((rows_per_blk, 128), lambda i: (i, 0))],
        out_specs=pl.BlockSpec((rows_per_blk, 128), lambda i: (i, 0)),
        out_shape=jax.ShapeDtypeStruct((rows, 128), jnp.float32),
    )(v2d)
    return out_idx, out.reshape(_NNZ)


# speedup vs baseline: 1.1411x; 1.1411x over previous
import functools

import jax
import jax.numpy as jnp
import numpy as np
from jax.experimental import pallas as pl

_NNZ = 6400000
_SCALE = np.float32(1.0 / (1.0 - 0.2 + 1e-05))

_K1 = np.uint32(0)
_K2 = np.uint32(42)
_K3 = np.uint32(0x1BD11BDA) ^ _K1 ^ _K2

_THRESH = np.uint32(1677722 << 9)

_ROT0 = (13, 15, 26, 6)
_ROT1 = (17, 29, 16, 24)


def _rotl(x, d):
    return (x << np.uint32(d)) | (x >> np.uint32(32 - d))


def _threefry_bits_from_x1(x1):
    ks = (_K1, _K2, _K3)
    x0 = x1
    x1 = x0 ^ _rotl(x1, _ROT0[0])
    first = True
    for r in range(5):
        rots = _ROT0 if r % 2 == 0 else _ROT1
        for d in rots:
            if first:
                first = False
                continue
            x0 = x0 + x1
            x1 = x0 ^ _rotl(x1, d)
        x0 = x0 + ks[(r + 1) % 3]
        x1 = x1 + ks[(r + 2) % 3] + np.uint32(r + 1)
    return x0 ^ x1


def _edge_drop_body(rows_per_blk, v_ref, a_ref, i_ref, o_ref, oi_ref):
    pid = pl.program_id(0)
    base = (pid * np.int32(rows_per_blk * 128)).astype(jnp.uint32)
    x1 = a_ref[...] + base
    bits = _threefry_bits_from_x1(x1)
    keep = bits >= _THRESH
    o_ref[...] = jnp.where(keep, v_ref[...] * _SCALE, jnp.float32(0.0))
    oi_ref[...] = i_ref[...]


def kernel(x_indices, x_values):
    rows = 50000
    rows_per_blk = 2000
    grid = rows // rows_per_blk
    icols = _NNZ // grid
    v2d = x_values.reshape(rows, 128)
    offs = jnp.asarray(
        np.arange(rows_per_blk * 128, dtype=np.uint32).reshape(rows_per_blk, 128)
        + _K2
    )
    out, out_idx = pl.pallas_call(
        functools.partial(_edge_drop_body, rows_per_blk),
        grid=(grid,),
        in_specs=[
            pl.BlockSpec((rows_per_blk, 128), lambda i: (i, 0)),
            pl.BlockSpec((rows_per_blk, 128), lambda i: (0, 0)),
            pl.BlockSpec((2, icols), lambda i: (0, i)),
        ],
        out_specs=[
            pl.BlockSpec((rows_per_blk, 128), lambda i: (i, 0)),
            pl.BlockSpec((2, icols), lambda i: (0, i)),
        ],
        out_shape=[
            jax.ShapeDtypeStruct((rows, 128), jnp.float32),
            jax.ShapeDtypeStruct((2, _NNZ), jnp.int32),
        ],
    )(v2d, offs, x_indices)
    return out_idx, out.reshape(_NNZ)


# speedup vs baseline: 1.1595x; 1.0161x over previous
import functools

import jax
import jax.numpy as jnp
import numpy as np
from jax.experimental import pallas as pl

_NNZ = 6400000
_SCALE = np.float32(1.0 / (1.0 - 0.2 + 1e-05))

_K1 = np.uint32(0)
_K2 = np.uint32(42)
_K3 = np.uint32(0x1BD11BDA) ^ _K1 ^ _K2

_THRESH = np.uint32(1677722 << 9)

_ROT0 = (13, 15, 26, 6)
_ROT1 = (17, 29, 16, 24)


def _rotl(x, d):
    return (x << np.uint32(d)) | (x >> np.uint32(32 - d))


def _threefry_bits_from_x1(x1):
    ks = (_K1, _K2, _K3)
    x0 = x1
    x1 = x0 ^ _rotl(x1, _ROT0[0])
    first = True
    for r in range(5):
        rots = _ROT0 if r % 2 == 0 else _ROT1
        for d in rots:
            if first:
                first = False
                continue
            x0 = x0 + x1
            x1 = x0 ^ _rotl(x1, d)
        x0 = x0 + ks[(r + 1) % 3]
        x1 = x1 + ks[(r + 2) % 3] + np.uint32(r + 1)
    return x0 ^ x1


def _edge_drop_body(rows_per_blk, v_ref, a_ref, i_ref, o_ref, oi_ref):
    x1 = a_ref[...]
    bits = _threefry_bits_from_x1(x1)
    keep = bits >= _THRESH
    o_ref[...] = jnp.where(keep, v_ref[...] * _SCALE, jnp.float32(0.0))
    oi_ref[...] = i_ref[...]


def kernel(x_indices, x_values):
    rows = 50000
    rows_per_blk = 2000
    grid = rows // rows_per_blk
    icols = _NNZ // grid
    v2d = x_values.reshape(rows, 128)
    offs = jnp.asarray(
        np.arange(_NNZ, dtype=np.uint32).reshape(rows, 128) + _K2
    )
    out, out_idx = pl.pallas_call(
        functools.partial(_edge_drop_body, rows_per_blk),
        grid=(grid,),
        in_specs=[
            pl.BlockSpec((rows_per_blk, 128), lambda i: (i, 0)),
            pl.BlockSpec((rows_per_blk, 128), lambda i: (i, 0)),
            pl.BlockSpec((2, icols), lambda i: (0, i)),
        ],
        out_specs=[
            pl.BlockSpec((rows_per_blk, 128), lambda i: (i, 0)),
            pl.BlockSpec((2, icols), lambda i: (0, i)),
        ],
        out_shape=[
            jax.ShapeDtypeStruct((rows, 128), jnp.float32),
            jax.ShapeDtypeStruct((2, _NNZ), jnp.int32),
        ],
    )(v2d, offs, x_indices)
    return out_idx, out.reshape(_NNZ)
